# baseline (speedup 1.0000x reference)
import functools

import jax
import jax.numpy as jnp
from jax import lax
from jax.experimental import pallas as pl
from jax.experimental.pallas import tpu as pltpu
from jax.experimental.pallas import tpu_sc as plsc

INPUT_DIM = 64
HIDDEN = 256
B = 16
S = 20
T = 20
VTILE = 2048

_NTOK = B * S
_PAD = 512
_BPW = _PAD // 32


def _sc_gather2(src2, tgt2, idx2_src, idx2_tgt):
    mesh = plsc.VectorSubcoreMesh(core_axis_name="c", subcore_axis_name="s")

    @functools.partial(
        pl.kernel,
        mesh=mesh,
        out_type=(jax.ShapeDtypeStruct((_PAD, 2 * INPUT_DIM), jnp.float32),
                  jax.ShapeDtypeStruct((_PAD, 2 * INPUT_DIM), jnp.float32)),
        scratch_types=[
            pltpu.VMEM((_BPW,), jnp.int32),
            pltpu.VMEM((_BPW, 2 * INPUT_DIM), jnp.float32),
            pltpu.VMEM((_BPW,), jnp.int32),
            pltpu.VMEM((_BPW, 2 * INPUT_DIM), jnp.float32),
            pltpu.SemaphoreType.DMA,
            pltpu.SemaphoreType.DMA,
        ],
    )
    def gather_k(src_hbm, tgt_hbm, isrc_hbm, itgt_hbm, out_src, out_tgt,
                 iv1, rv1, iv2, rv2, sem1, sem2):
        wid = lax.axis_index("s") * 2 + lax.axis_index("c")
        base = wid * _BPW
        pltpu.sync_copy(isrc_hbm.at[pl.ds(base, _BPW)], iv1)
        pltpu.sync_copy(itgt_hbm.at[pl.ds(base, _BPW)], iv2)
        h1 = pltpu.async_copy(src_hbm.at[iv1], rv1, sem1)
        h2 = pltpu.async_copy(tgt_hbm.at[iv2], rv2, sem2)
        h1.wait()
        h2.wait()
        pltpu.sync_copy(rv1, out_src.at[pl.ds(base, _BPW)])
        pltpu.sync_copy(rv2, out_tgt.at[pl.ds(base, _BPW)])

    return gather_k(src2, tgt2, idx2_src, idx2_tgt)


def _matmul_t(a, b):
    return lax.dot_general(a, b, (((1,), (1,)), ((), ())),
                           preferred_element_type=jnp.float32)


def _seq2seq_body(src_ref, tgt_ref, psrc_ref, ptgt_ref,
                  ew0i, ew0h, eb0, ew1i, ew1h, eb1,
                  dw0i, dw0h, db0, dw1i, dw1h, db1, wout_ref, bout_ref,
                  out_ref, y_ref):
    @pl.when(pl.program_id(0) == 0)
    def _prologue():
        def layer(x_seq, wih_r, whh_r, b_r, h, c):
            whh = whh_r[...]
            xw = _matmul_t(x_seq, wih_r[...]) + b_r[...]
            outs = []
            for t in range(T):
                z = xw[t * B:(t + 1) * B] + _matmul_t(h, whh)
                zi = z[:, :HIDDEN]
                zf = z[:, HIDDEN:2 * HIDDEN]
                zg = z[:, 2 * HIDDEN:3 * HIDDEN]
                zo = z[:, 3 * HIDDEN:]
                c = jax.nn.sigmoid(zf) * c + jax.nn.sigmoid(zi) * jnp.tanh(zg)
                h = jax.nn.sigmoid(zo) * jnp.tanh(c)
                outs.append(h)
            return outs, h, c

        def half_select(g_ref, p_ref):
            g = g_ref[...]
            return jnp.where(p_ref[...] > 0.5,
                             g[:, INPUT_DIM:], g[:, :INPUT_DIM])

        zeros = jnp.zeros((B, HIDDEN), jnp.float32)
        e0, h0, c0 = layer(half_select(src_ref, psrc_ref),
                           ew0i, ew0h, eb0, zeros, zeros)
        _, h1, c1 = layer(jnp.concatenate(e0, axis=0), ew1i, ew1h, eb1,
                          zeros, zeros)
        d0, _, _ = layer(half_select(tgt_ref, ptgt_ref),
                         dw0i, dw0h, db0, h0, c0)
        d1, _, _ = layer(jnp.concatenate(d0, axis=0), dw1i, dw1h, db1,
                         h1, c1)
        rows = []
        for b in range(B):
            rows.append(jnp.concatenate([d1[t][b:b + 1, :] for t in range(T)],
                                        axis=0))
        y_ref[...] = jnp.concatenate(rows, axis=0)

    out_ref[...] = _matmul_t(y_ref[...], wout_ref[...]) + bout_ref[...]


def kernel(input_sequence, target_sequence, src_table, tgt_table,
           enc_W_ih_0, enc_W_hh_0, enc_b_0, enc_W_ih_1, enc_W_hh_1, enc_b_1,
           dec_W_ih_0, dec_W_hh_0, dec_b_0, dec_W_ih_1, dec_W_hh_1, dec_b_1,
           W_out, b_out):
    V = W_out.shape[0]
    n_tiles = pl.cdiv(V, VTILE)

    idx_src = input_sequence.T.reshape(-1)
    idx_tgt = target_sequence.T.reshape(-1)
    pad = jnp.zeros((_PAD - _NTOK,), jnp.int32)
    i2s = jnp.concatenate([jnp.right_shift(idx_src, 1), pad])
    i2t = jnp.concatenate([jnp.right_shift(idx_tgt, 1), pad])
    p_src = jnp.bitwise_and(idx_src, 1).astype(jnp.float32).reshape(-1, 1)
    p_tgt = jnp.bitwise_and(idx_tgt, 1).astype(jnp.float32).reshape(-1, 1)
    g_src, g_tgt = _sc_gather2(
        src_table.reshape(-1, 2 * INPUT_DIM),
        tgt_table.reshape(-1, 2 * INPUT_DIM), i2s, i2t)
    src_emb = g_src[:_NTOK]
    tgt_emb = g_tgt[:_NTOK]

    full = lambda shape: pl.BlockSpec(shape, lambda i: (0,) * len(shape))
    logits = pl.pallas_call(
        _seq2seq_body,
        grid=(n_tiles,),
        in_specs=[
            full((S * B, 2 * INPUT_DIM)),
            full((T * B, 2 * INPUT_DIM)),
            full((S * B, 1)),
            full((T * B, 1)),
            full((4 * HIDDEN, INPUT_DIM)),
            full((4 * HIDDEN, HIDDEN)),
            full((1, 4 * HIDDEN)),
            full((4 * HIDDEN, HIDDEN)),
            full((4 * HIDDEN, HIDDEN)),
            full((1, 4 * HIDDEN)),
            full((4 * HIDDEN, INPUT_DIM)),
            full((4 * HIDDEN, HIDDEN)),
            full((1, 4 * HIDDEN)),
            full((4 * HIDDEN, HIDDEN)),
            full((4 * HIDDEN, HIDDEN)),
            full((1, 4 * HIDDEN)),
            pl.BlockSpec((VTILE, HIDDEN), lambda i: (i, 0)),
            pl.BlockSpec((1, VTILE), lambda i: (0, i)),
        ],
        out_specs=pl.BlockSpec((T * B, VTILE), lambda i: (0, i)),
        out_shape=jax.ShapeDtypeStruct((T * B, V), jnp.float32),
        scratch_shapes=[pltpu.VMEM((T * B, HIDDEN), jnp.float32)],
    )(src_emb, tgt_emb, p_src, p_tgt,
      enc_W_ih_0, enc_W_hh_0, enc_b_0.reshape(1, -1),
      enc_W_ih_1, enc_W_hh_1, enc_b_1.reshape(1, -1),
      dec_W_ih_0, dec_W_hh_0, dec_b_0.reshape(1, -1),
      dec_W_ih_1, dec_W_hh_1, dec_b_1.reshape(1, -1),
      W_out, b_out.reshape(1, -1))
    return logits.reshape(B, T, V)


# speedup vs baseline: 1.1149x; 1.1149x over previous
import jax
import jax.numpy as jnp
from jax import lax
from jax.experimental import pallas as pl
from jax.experimental.pallas import tpu as pltpu

INPUT_DIM = 64
HIDDEN = 256
B = 16
S = 20
T = 20
NTOK = B * S
VTILE = 2048
_CHUNK = 64


def _matmul_t(a, b):
    return lax.dot_general(a, b, (((1,), (1,)), ((), ())),
                           preferred_element_type=jnp.float32)


def _seq2seq_body(idx_s_ref, idx_t_ref, src_hbm, tgt_hbm,
                  ew0i, ew0h, eb0, ew1i, ew1h, eb1,
                  dw0i, dw0h, db0, dw1i, dw1h, db1, wout_ref, bout_ref,
                  out_ref, es_ref, et_ref, y_ref, sem_s, sem_t):
    @pl.when(pl.program_id(0) == 0)
    def _prologue():
        def gather(table_hbm, idx_ref, emb_ref, sem):
            for c0 in range(0, NTOK, _CHUNK):
                handles = []
                for i in range(c0, c0 + _CHUNK):
                    handles.append(pltpu.make_async_copy(
                        table_hbm.at[pl.ds(idx_ref[i], 1)],
                        emb_ref.at[pl.ds(i, 1)], sem))
                for h in handles:
                    h.start()
                for h in handles:
                    h.wait()

        gather(src_hbm, idx_s_ref, es_ref, sem_s)
        gather(tgt_hbm, idx_t_ref, et_ref, sem_t)

        def layer(x_seq, wih_r, whh_r, b_r, h, c):
            whh = whh_r[...]
            xw = _matmul_t(x_seq, wih_r[...]) + b_r[...]
            outs = []
            for t in range(T):
                z = xw[t * B:(t + 1) * B] + _matmul_t(h, whh)
                zi = z[:, :HIDDEN]
                zf = z[:, HIDDEN:2 * HIDDEN]
                zg = z[:, 2 * HIDDEN:3 * HIDDEN]
                zo = z[:, 3 * HIDDEN:]
                c = jax.nn.sigmoid(zf) * c + jax.nn.sigmoid(zi) * jnp.tanh(zg)
                h = jax.nn.sigmoid(zo) * jnp.tanh(c)
                outs.append(h)
            return outs, h, c

        zeros = jnp.zeros((B, HIDDEN), jnp.float32)
        e0, h0, c0 = layer(es_ref[...], ew0i, ew0h, eb0, zeros, zeros)
        _, h1, c1 = layer(jnp.concatenate(e0, axis=0), ew1i, ew1h, eb1,
                          zeros, zeros)
        d0, _, _ = layer(et_ref[...], dw0i, dw0h, db0, h0, c0)
        d1, _, _ = layer(jnp.concatenate(d0, axis=0), dw1i, dw1h, db1,
                         h1, c1)
        rows = []
        for b in range(B):
            rows.append(jnp.concatenate([d1[t][b:b + 1, :] for t in range(T)],
                                        axis=0))
        y_ref[...] = jnp.concatenate(rows, axis=0)

    out_ref[...] = _matmul_t(y_ref[...], wout_ref[...]) + bout_ref[...]


def kernel(input_sequence, target_sequence, src_table, tgt_table,
           enc_W_ih_0, enc_W_hh_0, enc_b_0, enc_W_ih_1, enc_W_hh_1, enc_b_1,
           dec_W_ih_0, dec_W_hh_0, dec_b_0, dec_W_ih_1, dec_W_hh_1, dec_b_1,
           W_out, b_out):
    V = W_out.shape[0]
    n_tiles = pl.cdiv(V, VTILE)

    idx_src = input_sequence.T.reshape(-1)
    idx_tgt = target_sequence.T.reshape(-1)

    full = lambda shape: pl.BlockSpec(shape, lambda i, *_: (0,) * len(shape))
    hbm = pl.BlockSpec(memory_space=pltpu.MemorySpace.HBM)
    grid_spec = pltpu.PrefetchScalarGridSpec(
        num_scalar_prefetch=2,
        grid=(n_tiles,),
        in_specs=[
            hbm,
            hbm,
            full((4 * HIDDEN, INPUT_DIM)),
            full((4 * HIDDEN, HIDDEN)),
            full((1, 4 * HIDDEN)),
            full((4 * HIDDEN, HIDDEN)),
            full((4 * HIDDEN, HIDDEN)),
            full((1, 4 * HIDDEN)),
            full((4 * HIDDEN, INPUT_DIM)),
            full((4 * HIDDEN, HIDDEN)),
            full((1, 4 * HIDDEN)),
            full((4 * HIDDEN, HIDDEN)),
            full((4 * HIDDEN, HIDDEN)),
            full((1, 4 * HIDDEN)),
            pl.BlockSpec((VTILE, HIDDEN), lambda i, *_: (i, 0)),
            pl.BlockSpec((1, VTILE), lambda i, *_: (0, i)),
        ],
        out_specs=pl.BlockSpec((NTOK, VTILE), lambda i, *_: (0, i)),
        scratch_shapes=[
            pltpu.VMEM((NTOK, INPUT_DIM), jnp.float32),
            pltpu.VMEM((NTOK, INPUT_DIM), jnp.float32),
            pltpu.VMEM((NTOK, HIDDEN), jnp.float32),
            pltpu.SemaphoreType.DMA,
            pltpu.SemaphoreType.DMA,
        ],
    )
    logits = pl.pallas_call(
        _seq2seq_body,
        grid_spec=grid_spec,
        out_shape=jax.ShapeDtypeStruct((NTOK, V), jnp.float32),
    )(idx_src, idx_tgt, src_table, tgt_table,
      enc_W_ih_0, enc_W_hh_0, enc_b_0.reshape(1, -1),
      enc_W_ih_1, enc_W_hh_1, enc_b_1.reshape(1, -1),
      dec_W_ih_0, dec_W_hh_0, dec_b_0.reshape(1, -1),
      dec_W_ih_1, dec_W_hh_1, dec_b_1.reshape(1, -1),
      W_out, b_out.reshape(1, -1))
    return logits.reshape(B, T, V)


# speedup vs baseline: 1.6779x; 1.5050x over previous
import jax
import jax.numpy as jnp
from jax import lax
from jax.experimental import pallas as pl
from jax.experimental.pallas import tpu as pltpu

INPUT_DIM = 64
HIDDEN = 256
B = 16
S = 20
T = 20
NTOK = B * S
VTILE = 2048
_CHUNK = 64


def _matmul_t(a, b):
    return lax.dot_general(a, b, (((1,), (1,)), ((), ())),
                           preferred_element_type=jnp.float32)


def _seq2seq_body(idx_s_ref, idx_t_ref, src_hbm, tgt_hbm,
                  ew0i, ew0h, eb0, ew1i, ew1h, eb1,
                  dw0i, dw0h, db0, dw1i, dw1h, db1, wout_ref, bout_ref,
                  out_ref, es_ref, et_ref, y_ref, sem_s, sem_t):
    @pl.when(pl.program_id(0) == 0)
    def _prologue():
        def gather(table_hbm, idx_ref, emb_ref, sem):
            for c0 in range(0, NTOK, _CHUNK):
                handles = []
                for i in range(c0, c0 + _CHUNK):
                    handles.append(pltpu.make_async_copy(
                        table_hbm.at[pl.ds(idx_ref[i], 1)],
                        emb_ref.at[pl.ds(i, 1)], sem))
                for h in handles:
                    h.start()
                for h in handles:
                    h.wait()

        gather(src_hbm, idx_s_ref, es_ref, sem_s)
        gather(tgt_hbm, idx_t_ref, et_ref, sem_t)

        def layer(x_seq, wih_r, whh_r, b_r, h, c):
            whh = whh_r[...]
            xw = _matmul_t(x_seq, wih_r[...]) + b_r[...]
            outs = []
            for t in range(T):
                z = xw[t * B:(t + 1) * B] + _matmul_t(h, whh)
                zi = z[:, :HIDDEN]
                zf = z[:, HIDDEN:2 * HIDDEN]
                zg = z[:, 2 * HIDDEN:3 * HIDDEN]
                zo = z[:, 3 * HIDDEN:]
                c = jax.nn.sigmoid(zf) * c + jax.nn.sigmoid(zi) * jnp.tanh(zg)
                h = jax.nn.sigmoid(zo) * jnp.tanh(c)
                outs.append(h)
            return outs, h, c

        zeros = jnp.zeros((B, HIDDEN), jnp.float32)
        e0, h0, c0 = layer(es_ref[...], ew0i, ew0h, eb0, zeros, zeros)
        _, h1, c1 = layer(jnp.concatenate(e0, axis=0), ew1i, ew1h, eb1,
                          zeros, zeros)
        d0, _, _ = layer(et_ref[...], dw0i, dw0h, db0, h0, c0)
        d1, _, _ = layer(jnp.concatenate(d0, axis=0), dw1i, dw1h, db1,
                         h1, c1)
        rows = []
        for b in range(B):
            rows.append(jnp.concatenate([d1[t][b:b + 1, :] for t in range(T)],
                                        axis=0))
        y_ref[...] = jnp.concatenate(rows, axis=0)

    logits = _matmul_t(y_ref[...], wout_ref[...]) + bout_ref[...]
    for b in range(B):
        out_ref[b] = logits[b * T:(b + 1) * T]


def kernel(input_sequence, target_sequence, src_table, tgt_table,
           enc_W_ih_0, enc_W_hh_0, enc_b_0, enc_W_ih_1, enc_W_hh_1, enc_b_1,
           dec_W_ih_0, dec_W_hh_0, dec_b_0, dec_W_ih_1, dec_W_hh_1, dec_b_1,
           W_out, b_out):
    V = W_out.shape[0]
    n_tiles = pl.cdiv(V, VTILE)

    idx_src = input_sequence.T.reshape(-1)
    idx_tgt = target_sequence.T.reshape(-1)

    full = lambda shape: pl.BlockSpec(shape, lambda i, *_: (0,) * len(shape))
    hbm = pl.BlockSpec(memory_space=pltpu.MemorySpace.HBM)
    grid_spec = pltpu.PrefetchScalarGridSpec(
        num_scalar_prefetch=2,
        grid=(n_tiles,),
        in_specs=[
            hbm,
            hbm,
            full((4 * HIDDEN, INPUT_DIM)),
            full((4 * HIDDEN, HIDDEN)),
            full((1, 4 * HIDDEN)),
            full((4 * HIDDEN, HIDDEN)),
            full((4 * HIDDEN, HIDDEN)),
            full((1, 4 * HIDDEN)),
            full((4 * HIDDEN, INPUT_DIM)),
            full((4 * HIDDEN, HIDDEN)),
            full((1, 4 * HIDDEN)),
            full((4 * HIDDEN, HIDDEN)),
            full((4 * HIDDEN, HIDDEN)),
            full((1, 4 * HIDDEN)),
            pl.BlockSpec((VTILE, HIDDEN), lambda i, *_: (i, 0)),
            pl.BlockSpec((1, VTILE), lambda i, *_: (0, i)),
        ],
        out_specs=pl.BlockSpec((B, T, VTILE), lambda i, *_: (0, 0, i)),
        scratch_shapes=[
            pltpu.VMEM((NTOK, INPUT_DIM), jnp.float32),
            pltpu.VMEM((NTOK, INPUT_DIM), jnp.float32),
            pltpu.VMEM((NTOK, HIDDEN), jnp.float32),
            pltpu.SemaphoreType.DMA,
            pltpu.SemaphoreType.DMA,
        ],
    )
    return pl.pallas_call(
        _seq2seq_body,
        grid_spec=grid_spec,
        out_shape=jax.ShapeDtypeStruct((B, T, V), jnp.float32),
    )(idx_src, idx_tgt, src_table, tgt_table,
      enc_W_ih_0, enc_W_hh_0, enc_b_0.reshape(1, -1),
      enc_W_ih_1, enc_W_hh_1, enc_b_1.reshape(1, -1),
      dec_W_ih_0, dec_W_hh_0, dec_b_0.reshape(1, -1),
      dec_W_ih_1, dec_W_hh_1, dec_b_1.reshape(1, -1),
      W_out, b_out.reshape(1, -1))
